# Initial kernel scaffold; baseline (speedup 1.0000x reference)
import jax
import jax.numpy as jnp
from jax.experimental import pallas as pl


def kernel(inputs, symbols, latents):
    raise NotImplementedError("write your pallas kernel here")


# speedup vs baseline: 1.4310x; 1.4310x over previous
import functools

import jax
import jax.numpy as jnp
from jax import lax
from jax.experimental import pallas as pl
from jax.experimental.pallas import tpu as pltpu
from jax.experimental.pallas import tpu_sc as plsc

_NUM_CORES = 2
_NUM_SUBCORES = 16
_NUM_WORKERS = _NUM_CORES * _NUM_SUBCORES


def _sc_gather(table, idx, chunk):
    n, d = idx.shape[0], table.shape[1]
    per_worker = n // _NUM_WORKERS
    n_chunks = per_worker // chunk
    assert per_worker * _NUM_WORKERS == n and n_chunks * chunk == per_worker
    assert chunk % 8 == 0 and chunk <= 128

    mesh = plsc.VectorSubcoreMesh(core_axis_name="c", subcore_axis_name="s")

    @functools.partial(
        pl.kernel,
        mesh=mesh,
        out_type=jax.ShapeDtypeStruct((n, d), jnp.float32),
        scratch_types=[
            pltpu.VMEM((chunk,), jnp.int32),
            pltpu.VMEM((chunk,), jnp.int32),
            pltpu.VMEM((chunk, d), jnp.float32),
            pltpu.VMEM((chunk, d), jnp.float32),
            pltpu.SemaphoreType.DMA,
            pltpu.SemaphoreType.DMA,
        ],
    )
    def gather_kernel(table_hbm, idx_hbm, out_hbm, idx_a, idx_b, rows_a,
                      rows_b, sem_a, sem_b):
        wid = lax.axis_index("s") * _NUM_CORES + lax.axis_index("c")
        base = wid * per_worker
        idx_v = [idx_a, idx_b]
        rows_v = [rows_a, rows_b]
        sems = [sem_a, sem_b]

        pltpu.sync_copy(idx_hbm.at[pl.ds(base, chunk)], idx_a)
        gather0 = pltpu.async_copy(table_hbm.at[idx_a], rows_a, sem_a)
        for c in range(n_chunks):
            cur = c % 2
            nxt = (c + 1) % 2
            if c + 1 < n_chunks:
                off = base + (c + 1) * chunk
                pltpu.sync_copy(idx_hbm.at[pl.ds(off, chunk)], idx_v[nxt])
                pltpu.async_copy(table_hbm.at[idx_v[nxt]], rows_v[nxt],
                                 sems[nxt])
            if c == 0:
                gather0.wait()
            else:
                pltpu.make_async_copy(table_hbm.at[idx_v[cur]], rows_v[cur],
                                      sems[cur]).wait()
            pltpu.sync_copy(rows_v[cur],
                            out_hbm.at[pl.ds(base + c * chunk, chunk)])

    return gather_kernel(table, idx)


def _vq_body(x_ref, s_ref, o_ref):
    x = x_ref[...]
    s = s_ref[...]
    xs = lax.dot_general(x, s, (((1,), (1,)), ((), ())),
                         preferred_element_type=jnp.float32)
    s2 = lax.dot_general(jnp.ones((1, s.shape[1]), jnp.float32), s * s,
                         (((1,), (1,)), ((), ())),
                         preferred_element_type=jnp.float32,
                         precision=lax.Precision.HIGHEST)
    dists = s2 - 2.0 * xs
    m = jnp.min(dists, axis=1, keepdims=True)
    k_iota = lax.broadcasted_iota(jnp.int32, dists.shape, 1)
    k = s.shape[0]
    idx = jnp.min(jnp.where(dists == m, k_iota, jnp.int32(k)),
                  axis=1, keepdims=True)
    one_hot = (idx == k_iota).astype(jnp.float32)
    o_ref[...] = lax.dot_general(one_hot, s, (((1,), (0,)), ((), ())),
                                 preferred_element_type=jnp.float32,
                                 precision=lax.Precision.HIGHEST)


def _vq_lookup(x, symbols, block):
    m, d = x.shape
    k = symbols.shape[0]
    assert m % block == 0
    return pl.pallas_call(
        _vq_body,
        grid=(m // block,),
        in_specs=[
            pl.BlockSpec((block, d), lambda i: (i, 0)),
            pl.BlockSpec((k, d), lambda i: (0, 0)),
        ],
        out_specs=pl.BlockSpec((block, d), lambda i: (i, 0)),
        out_shape=jax.ShapeDtypeStruct((m, d), jnp.float32),
    )(x, symbols)


def kernel(inputs, symbols, latents):
    b, l = inputs.shape
    v, p, d = latents.shape
    n = b * l
    table = latents.reshape(v, p * d)
    idx = inputs.reshape(n).astype(jnp.int32)
    gathered = _sc_gather(table, idx, chunk=104)
    quantized = _vq_lookup(gathered.reshape(n * p, d), symbols, block=2048)
    return quantized.reshape(b, l, p * d)


# speedup vs baseline: 2.1150x; 1.4780x over previous
import functools

import jax
import jax.numpy as jnp
from jax import lax
from jax.experimental import pallas as pl
from jax.experimental.pallas import tpu as pltpu
from jax.experimental.pallas import tpu_sc as plsc

_NUM_CORES = 2
_NUM_SUBCORES = 16
_NUM_WORKERS = _NUM_CORES * _NUM_SUBCORES


def _sc_gather(table, idx, chunk):
    n, d = idx.shape[0], table.shape[1]
    per_worker = n // _NUM_WORKERS
    n_chunks = per_worker // chunk
    assert per_worker * _NUM_WORKERS == n and n_chunks * chunk == per_worker
    assert chunk % 8 == 0 and chunk <= 128

    mesh = plsc.VectorSubcoreMesh(core_axis_name="c", subcore_axis_name="s")

    @functools.partial(
        pl.kernel,
        mesh=mesh,
        out_type=jax.ShapeDtypeStruct((n, d), jnp.float32),
        scratch_types=[
            pltpu.VMEM((chunk,), jnp.int32),
            pltpu.VMEM((chunk,), jnp.int32),
            pltpu.VMEM((chunk, d), jnp.float32),
            pltpu.VMEM((chunk, d), jnp.float32),
            pltpu.SemaphoreType.DMA,
            pltpu.SemaphoreType.DMA,
        ],
    )
    def gather_kernel(table_hbm, idx_hbm, out_hbm, idx_a, idx_b, rows_a,
                      rows_b, sem_a, sem_b):
        wid = lax.axis_index("s") * _NUM_CORES + lax.axis_index("c")
        base = wid * per_worker
        idx_v = [idx_a, idx_b]
        rows_v = [rows_a, rows_b]
        sems = [sem_a, sem_b]

        pltpu.sync_copy(idx_hbm.at[pl.ds(base, chunk)], idx_a)
        gather0 = pltpu.async_copy(table_hbm.at[idx_a], rows_a, sem_a)
        for c in range(n_chunks):
            cur = c % 2
            nxt = (c + 1) % 2
            if c + 1 < n_chunks:
                off = base + (c + 1) * chunk
                pltpu.sync_copy(idx_hbm.at[pl.ds(off, chunk)], idx_v[nxt])
                pltpu.async_copy(table_hbm.at[idx_v[nxt]], rows_v[nxt],
                                 sems[nxt])
            if c == 0:
                gather0.wait()
            else:
                pltpu.make_async_copy(table_hbm.at[idx_v[cur]], rows_v[cur],
                                      sems[cur]).wait()
            pltpu.sync_copy(rows_v[cur],
                            out_hbm.at[pl.ds(base + c * chunk, chunk)])

    return gather_kernel(table, idx)


def _vq_body(x_ref, s_ref, o_ref):
    x = x_ref[...]
    s = s_ref[...]
    xs_n2 = lax.dot_general(x * -2.0, s, (((1,), (1,)), ((), ())),
                            preferred_element_type=jnp.float32)
    s2 = lax.dot_general(jnp.ones((1, s.shape[1]), jnp.float32), s * s,
                         (((1,), (1,)), ((), ())),
                         preferred_element_type=jnp.float32,
                         precision=lax.Precision.HIGHEST)
    dists = s2 + xs_n2
    m = jnp.min(dists, axis=1, keepdims=True)
    k_iota = lax.broadcasted_iota(jnp.int32, dists.shape, 1)
    k = s.shape[0]
    idx = jnp.min(jnp.where(dists == m, k_iota, jnp.int32(k)),
                  axis=1, keepdims=True)
    one_hot = (idx == k_iota).astype(jnp.float32)
    o_ref[...] = lax.dot_general(one_hot, s, (((1,), (0,)), ((), ())),
                                 preferred_element_type=jnp.float32)


def _vq_lookup(x, symbols, block):
    m, d = x.shape
    k = symbols.shape[0]
    assert m % block == 0
    return pl.pallas_call(
        _vq_body,
        grid=(m // block,),
        in_specs=[
            pl.BlockSpec((block, d), lambda i: (i, 0)),
            pl.BlockSpec((k, d), lambda i: (0, 0)),
        ],
        out_specs=pl.BlockSpec((block, d), lambda i: (i, 0)),
        out_shape=jax.ShapeDtypeStruct((m, d), jnp.float32),
    )(x, symbols)


def kernel(inputs, symbols, latents):
    b, l = inputs.shape
    v, p, d = latents.shape
    n = b * l
    table = latents.reshape(v, p * d)
    idx = jnp.transpose(inputs).reshape(n).astype(jnp.int32)
    gathered = _sc_gather(table, idx, chunk=104)
    quantized = _vq_lookup(gathered.reshape(n * p, d), symbols, block=2048)
    return jnp.transpose(quantized.reshape(l, b, p * d), (1, 0, 2))


# speedup vs baseline: 2.6836x; 1.2688x over previous
import functools

import jax
import jax.numpy as jnp
from jax import lax
from jax.experimental import pallas as pl
from jax.experimental.pallas import tpu as pltpu
from jax.experimental.pallas import tpu_sc as plsc

_NUM_CORES = 2
_NUM_SUBCORES = 16
_NUM_WORKERS = _NUM_CORES * _NUM_SUBCORES


def _sc_gather(table, idx, chunk):
    n, d = idx.shape[0], table.shape[1]
    per_worker = n // _NUM_WORKERS
    n_chunks = per_worker // chunk
    assert per_worker * _NUM_WORKERS == n and n_chunks * chunk == per_worker
    assert chunk % 8 == 0 and chunk <= 128

    mesh = plsc.VectorSubcoreMesh(core_axis_name="c", subcore_axis_name="s")

    @functools.partial(
        pl.kernel,
        mesh=mesh,
        out_type=jax.ShapeDtypeStruct((n, d), jnp.float32),
        scratch_types=[
            pltpu.VMEM((chunk,), jnp.int32),
            pltpu.VMEM((chunk,), jnp.int32),
            pltpu.VMEM((chunk, d), jnp.float32),
            pltpu.VMEM((chunk, d), jnp.float32),
            pltpu.SemaphoreType.DMA,
            pltpu.SemaphoreType.DMA,
        ],
    )
    def gather_kernel(table_hbm, idx_hbm, out_hbm, idx_a, idx_b, rows_a,
                      rows_b, sem_a, sem_b):
        wid = lax.axis_index("s") * _NUM_CORES + lax.axis_index("c")
        base = wid * per_worker
        idx_v = [idx_a, idx_b]
        rows_v = [rows_a, rows_b]
        sems = [sem_a, sem_b]

        pltpu.sync_copy(idx_hbm.at[pl.ds(base, chunk)], idx_a)
        gather0 = pltpu.async_copy(table_hbm.at[idx_a], rows_a, sem_a)
        for c in range(n_chunks):
            cur = c % 2
            nxt = (c + 1) % 2
            if c + 1 < n_chunks:
                off = base + (c + 1) * chunk
                pltpu.sync_copy(idx_hbm.at[pl.ds(off, chunk)], idx_v[nxt])
                pltpu.async_copy(table_hbm.at[idx_v[nxt]], rows_v[nxt],
                                 sems[nxt])
            if c == 0:
                gather0.wait()
            else:
                pltpu.make_async_copy(table_hbm.at[idx_v[cur]], rows_v[cur],
                                      sems[cur]).wait()
            pltpu.sync_copy(rows_v[cur],
                            out_hbm.at[pl.ds(base + c * chunk, chunk)])

    return gather_kernel(table, idx)


def _vq_body(x_ref, s_ref, o_ref):
    x = x_ref[...]
    s = s_ref[...]
    xs_n2 = lax.dot_general(x * -2.0, s, (((1,), (1,)), ((), ())),
                            preferred_element_type=jnp.float32)
    s2 = lax.dot_general(jnp.ones((1, s.shape[1]), jnp.float32), s * s,
                         (((1,), (1,)), ((), ())),
                         preferred_element_type=jnp.float32,
                         precision=lax.Precision.HIGHEST)
    dists = s2 + xs_n2
    m = jnp.min(dists, axis=1, keepdims=True)
    one_hot = (dists == m).astype(jnp.float32)
    kh = s.shape[0] // 2
    eq_fold = one_hot[:, :kh] + one_hot[:, kh:]
    s_fold = jnp.concatenate([s[:kh], s[kh:]], axis=1)
    out2 = lax.dot_general(eq_fold, s_fold, (((1,), (0,)), ((), ())),
                           preferred_element_type=jnp.float32)
    m_lo = jnp.min(dists[:, :kh], axis=1, keepdims=True)
    dd = s.shape[1]
    o_ref[...] = jnp.where(m_lo == m, out2[:, :dd], out2[:, dd:])


def _vq_lookup(x, symbols, block):
    m, d = x.shape
    k = symbols.shape[0]
    assert m % block == 0
    return pl.pallas_call(
        _vq_body,
        grid=(m // block,),
        in_specs=[
            pl.BlockSpec((block, d), lambda i: (i, 0)),
            pl.BlockSpec((k, d), lambda i: (0, 0)),
        ],
        out_specs=pl.BlockSpec((block, d), lambda i: (i, 0)),
        out_shape=jax.ShapeDtypeStruct((m, d), jnp.float32),
    )(x, symbols)


def kernel(inputs, symbols, latents):
    b, l = inputs.shape
    v, p, d = latents.shape
    n = b * l
    table = latents.reshape(v, p * d)
    idx = jnp.transpose(inputs).reshape(n).astype(jnp.int32)
    gathered = _sc_gather(table, idx, chunk=104)
    quantized = _vq_lookup(gathered.reshape(n * p, d), symbols, block=4096)
    return jnp.transpose(quantized.reshape(l, b, p * d), (1, 0, 2))


# speedup vs baseline: 2.7213x; 1.0140x over previous
import functools

import jax
import jax.numpy as jnp
from jax import lax
from jax.experimental import pallas as pl
from jax.experimental.pallas import tpu as pltpu
from jax.experimental.pallas import tpu_sc as plsc

_NUM_CORES = 2
_NUM_SUBCORES = 16
_NUM_WORKERS = _NUM_CORES * _NUM_SUBCORES


def _sc_gather(table, idx, chunk):
    n, d = idx.shape[0], table.shape[1]
    per_worker = n // _NUM_WORKERS
    n_chunks = per_worker // chunk
    assert per_worker * _NUM_WORKERS == n and n_chunks * chunk == per_worker
    assert chunk % 8 == 0 and chunk <= 128

    mesh = plsc.VectorSubcoreMesh(core_axis_name="c", subcore_axis_name="s")

    @functools.partial(
        pl.kernel,
        mesh=mesh,
        out_type=jax.ShapeDtypeStruct((n, d), jnp.float32),
        scratch_types=[
            pltpu.VMEM((chunk,), jnp.int32),
            pltpu.VMEM((chunk,), jnp.int32),
            pltpu.VMEM((chunk, d), jnp.float32),
            pltpu.VMEM((chunk, d), jnp.float32),
            pltpu.SemaphoreType.DMA,
            pltpu.SemaphoreType.DMA,
        ],
    )
    def gather_kernel(table_hbm, idx_hbm, out_hbm, idx_a, idx_b, rows_a,
                      rows_b, sem_a, sem_b):
        wid = lax.axis_index("s") * _NUM_CORES + lax.axis_index("c")
        base = wid * per_worker
        idx_v = [idx_a, idx_b]
        rows_v = [rows_a, rows_b]
        sems = [sem_a, sem_b]

        pltpu.sync_copy(idx_hbm.at[pl.ds(base, chunk)], idx_a)
        gather0 = pltpu.async_copy(table_hbm.at[idx_a], rows_a, sem_a)
        for c in range(n_chunks):
            cur = c % 2
            nxt = (c + 1) % 2
            if c + 1 < n_chunks:
                off = base + (c + 1) * chunk
                pltpu.sync_copy(idx_hbm.at[pl.ds(off, chunk)], idx_v[nxt])
                pltpu.async_copy(table_hbm.at[idx_v[nxt]], rows_v[nxt],
                                 sems[nxt])
            if c == 0:
                gather0.wait()
            else:
                pltpu.make_async_copy(table_hbm.at[idx_v[cur]], rows_v[cur],
                                      sems[cur]).wait()
            pltpu.sync_copy(rows_v[cur],
                            out_hbm.at[pl.ds(base + c * chunk, chunk)])

    return gather_kernel(table, idx)


def _vq_body(x_ref, s_ref, o_ref):
    x = x_ref[...]
    s = s_ref[...]
    xs_n2 = lax.dot_general(x * -2.0, s, (((1,), (1,)), ((), ())),
                            preferred_element_type=jnp.float32)
    s2 = lax.dot_general(jnp.ones((1, s.shape[1]), jnp.float32), s * s,
                         (((1,), (1,)), ((), ())),
                         preferred_element_type=jnp.float32,
                         precision=lax.Precision.HIGHEST)
    dists = s2 + xs_n2
    m = jnp.min(dists, axis=1, keepdims=True)
    one_hot = (dists == m).astype(jnp.bfloat16)
    kh = s.shape[0] // 2
    eq_fold = one_hot[:, :kh] + one_hot[:, kh:]
    s_fold = jnp.concatenate([s[:kh], s[kh:]],
                             axis=1).astype(jnp.bfloat16)
    out2 = lax.dot_general(eq_fold, s_fold, (((1,), (0,)), ((), ())),
                           preferred_element_type=jnp.float32)
    m_lo = jnp.min(dists[:, :kh], axis=1, keepdims=True)
    dd = s.shape[1]
    o_ref[...] = jnp.where(m_lo == m, out2[:, :dd], out2[:, dd:])


def _vq_lookup(x, symbols, block):
    m, d = x.shape
    k = symbols.shape[0]
    assert m % block == 0
    return pl.pallas_call(
        _vq_body,
        grid=(m // block,),
        in_specs=[
            pl.BlockSpec((block, d), lambda i: (i, 0)),
            pl.BlockSpec((k, d), lambda i: (0, 0)),
        ],
        out_specs=pl.BlockSpec((block, d), lambda i: (i, 0)),
        out_shape=jax.ShapeDtypeStruct((m, d), jnp.float32),
    )(x, symbols)


def kernel(inputs, symbols, latents):
    b, l = inputs.shape
    v, p, d = latents.shape
    n = b * l
    table = latents.reshape(v, p * d)
    idx = jnp.transpose(inputs).reshape(n).astype(jnp.int32)
    gathered = _sc_gather(table, idx, chunk=104)
    quantized = _vq_lookup(gathered.reshape(n * p, d), symbols, block=8192)
    return jnp.transpose(quantized.reshape(l, b, p * d), (1, 0, 2))
